# initial kernel scaffold (unmeasured)
import jax
import jax.numpy as jnp
from jax import lax
from jax.experimental import pallas as pl
from jax.experimental.pallas import tpu as pltpu

N_DEV = 4
SQ = 2048
SKV_SHARD = 2048
HQ = 8
DH = 128
DM = HQ * DH
SCALE = 0.08838834764831843
QBLK = 256
N_QBLK = SQ // QBLK
PACK = DM + 128


def kernel(x, Wq, K_ext, V_ext, Wo):
    x2 = x.reshape(SQ, DM)
    K2 = K_ext.reshape(SKV_SHARD, DM)
    V2 = V_ext.reshape(SKV_SHARD, DM)

    def body(x_ref, wq_ref, k_ref, v_ref, wo_ref, o_ref,
             qbuf, kbuf, vbuf, comm, acc, send_sems, recv_sems):
        my = lax.axis_index("i")
        left = (my - 1) % N_DEV
        right = (my + 1) % N_DEV
        koff = my * SKV_SHARD

        barrier_sem = pltpu.get_barrier_semaphore()
        for nbr in (left, right):
            pl.semaphore_signal(barrier_sem, inc=1, device_id=(nbr,),
                                device_id_type=pl.DeviceIdType.MESH)
        pl.semaphore_wait(barrier_sem, 2)

        xb = x_ref[...].astype(jnp.bfloat16)
        wqb = wq_ref[...].astype(jnp.bfloat16)
        q = lax.dot_general(xb, wqb, (((1,), (0,)), ((), ())),
                            preferred_element_type=jnp.float32)
        qbuf[...] = (q * SCALE).astype(jnp.bfloat16)
        kbuf[...] = k_ref[...].astype(jnp.bfloat16)
        vbuf[...] = v_ref[...].astype(jnp.bfloat16)

        def qblock(b, carry):
            qstart = b * QBLK
            qi = qstart + lax.broadcasted_iota(jnp.int32, (QBLK, SKV_SHARD), 0)
            ki = koff + lax.broadcasted_iota(jnp.int32, (QBLK, SKV_SHARD), 1)
            keep = (jnp.abs(qi - ki) <= 128) | (ki < 32) | (qi < 32)
            ls = []
            for h in range(HQ):
                qh = qbuf[pl.ds(qstart, QBLK), h * DH:(h + 1) * DH]
                kh = kbuf[:, h * DH:(h + 1) * DH]
                vh = vbuf[:, h * DH:(h + 1) * DH]
                s = lax.dot_general(qh, kh, (((1,), (1,)), ((), ())),
                                    preferred_element_type=jnp.float32)
                w = jnp.where(keep, jnp.exp(s), 0.0)
                wb = w.astype(jnp.bfloat16)
                num = lax.dot_general(wb, vh, (((1,), (0,)), ((), ())),
                                      preferred_element_type=jnp.float32)
                comm[0, pl.ds(qstart, QBLK), h * DH:(h + 1) * DH] = (
                    num.astype(jnp.bfloat16))
                ls.append(jnp.sum(w, axis=1, keepdims=True))
            lblk = jnp.concatenate(
                ls + [jnp.zeros((QBLK, 128 - HQ), jnp.float32)], axis=1)
            comm[0, pl.ds(qstart, QBLK), DM:PACK] = lblk.astype(jnp.bfloat16)
            return carry

        lax.fori_loop(0, N_QBLK, qblock, 0)

        acc[...] = comm[0].astype(jnp.float32)

        for hop in range(N_DEV - 1):
            send_slot = hop % 2
            recv_slot = (hop + 1) % 2
            rdma = pltpu.make_async_remote_copy(
                src_ref=comm.at[send_slot],
                dst_ref=comm.at[recv_slot],
                send_sem=send_sems.at[send_slot],
                recv_sem=recv_sems.at[recv_slot],
                device_id=(right,),
                device_id_type=pl.DeviceIdType.MESH,
            )
            rdma.start()
            rdma.wait()
            acc[...] = acc[...] + comm[recv_slot].astype(jnp.float32)

        parts = []
        for h in range(HQ):
            num_h = acc[:, h * DH:(h + 1) * DH]
            l_h = acc[:, DM + h:DM + h + 1]
            parts.append((num_h / l_h).astype(jnp.bfloat16))
        ctx = jnp.concatenate(parts, axis=1)
        wob = wo_ref[...].astype(jnp.bfloat16)
        o_ref[...] = lax.dot_general(ctx, wob, (((1,), (0,)), ((), ())),
                                     preferred_element_type=jnp.float32)

    out = pl.pallas_call(
        body,
        out_shape=jax.ShapeDtypeStruct((SQ, DM), jnp.float32),
        in_specs=[pl.BlockSpec(memory_space=pltpu.VMEM)] * 5,
        out_specs=pl.BlockSpec(memory_space=pltpu.VMEM),
        scratch_shapes=[
            pltpu.VMEM((SQ, DM), jnp.bfloat16),
            pltpu.VMEM((SKV_SHARD, DM), jnp.bfloat16),
            pltpu.VMEM((SKV_SHARD, DM), jnp.bfloat16),
            pltpu.VMEM((2, SQ, PACK), jnp.bfloat16),
            pltpu.VMEM((SQ, PACK), jnp.float32),
            pltpu.SemaphoreType.DMA((2,)),
            pltpu.SemaphoreType.DMA((2,)),
        ],
        compiler_params=pltpu.CompilerParams(collective_id=0),
    )(x2, Wq, K2, V2, Wo)
    return out.reshape(1, SQ, DM)


# baseline (device time: 279423 ns/iter reference)
import jax
import jax.numpy as jnp
from jax import lax
from jax.experimental import pallas as pl
from jax.experimental.pallas import tpu as pltpu

N_DEV = 4
SQ = 2048
SKV_SHARD = 2048
HQ = 8
DH = 128
DM = HQ * DH
SCALE = 0.08838834764831843
QBLK = 128
N_QBLK = SQ // QBLK
PACK = DM + 128


def kernel(x, Wq, K_ext, V_ext, Wo):
    xb = x.reshape(SQ, DM).astype(jnp.bfloat16)
    Kb = K_ext.reshape(SKV_SHARD, DM).astype(jnp.bfloat16)
    Vb = V_ext.reshape(SKV_SHARD, DM).astype(jnp.bfloat16)
    Wqb = Wq.astype(jnp.bfloat16)
    Wob = Wo.astype(jnp.bfloat16)

    def body(x_ref, wq_ref, k_ref, v_ref, wo_ref, o_ref,
             comm, lacc, send_sems, recv_sems):
        my = lax.axis_index("i")
        left = (my - 1) % N_DEV
        right = (my + 1) % N_DEV
        koff = my * SKV_SHARD

        barrier_sem = pltpu.get_barrier_semaphore()
        for nbr in (left, right):
            pl.semaphore_signal(barrier_sem, inc=1, device_id=(nbr,),
                                device_id_type=pl.DeviceIdType.MESH)
        pl.semaphore_wait(barrier_sem, 2)

        def qblock(b, carry):
            qstart = b * QBLK
            q_blk = lax.dot_general(
                x_ref[pl.ds(qstart, QBLK), :], wq_ref[...],
                (((1,), (0,)), ((), ())),
                preferred_element_type=jnp.float32)
            q_blk = (q_blk * SCALE).astype(jnp.bfloat16)

            qi = qstart + lax.broadcasted_iota(jnp.int32, (QBLK, SKV_SHARD), 0)
            ki = koff + lax.broadcasted_iota(jnp.int32, (QBLK, SKV_SHARD), 1)
            keep = (jnp.abs(qi - ki) <= 128) | (ki < 32) | (qi < 32)
            ls = []
            for h in range(HQ):
                qh = q_blk[:, h * DH:(h + 1) * DH]
                kh = k_ref[:, h * DH:(h + 1) * DH]
                vh = v_ref[:, h * DH:(h + 1) * DH]
                s = lax.dot_general(qh, kh, (((1,), (1,)), ((), ())),
                                    preferred_element_type=jnp.float32)
                w = jnp.where(keep, jnp.exp(s), 0.0)
                wb = w.astype(jnp.bfloat16)
                num = lax.dot_general(wb, vh, (((1,), (0,)), ((), ())),
                                      preferred_element_type=jnp.float32)
                comm[0, pl.ds(qstart, QBLK), h * DH:(h + 1) * DH] = (
                    num.astype(jnp.bfloat16))
                ls.append(jnp.sum(w, axis=1, keepdims=True))
            lblk = jnp.concatenate(
                ls + [jnp.zeros((QBLK, 128 - HQ), jnp.float32)], axis=1)
            comm[0, pl.ds(qstart, QBLK), DM:PACK] = lblk.astype(jnp.bfloat16)
            return carry

        lax.fori_loop(0, N_QBLK, qblock, 0)

        o_ref[...] = comm[0, :, :DM].astype(jnp.float32)
        lacc[...] = comm[0, :, DM:].astype(jnp.float32)

        for hop in range(N_DEV - 1):
            send_slot = hop % 2
            recv_slot = (hop + 1) % 2
            rdma = pltpu.make_async_remote_copy(
                src_ref=comm.at[send_slot],
                dst_ref=comm.at[recv_slot],
                send_sem=send_sems.at[send_slot],
                recv_sem=recv_sems.at[recv_slot],
                device_id=(right,),
                device_id_type=pl.DeviceIdType.MESH,
            )
            rdma.start()
            rdma.wait()
            o_ref[...] = o_ref[...] + comm[recv_slot, :, :DM].astype(jnp.float32)
            lacc[...] = lacc[...] + comm[recv_slot, :, DM:].astype(jnp.float32)

        parts = []
        for h in range(HQ):
            num_h = o_ref[:, h * DH:(h + 1) * DH]
            l_h = lacc[:, h:h + 1]
            parts.append((num_h / l_h).astype(jnp.bfloat16))
        ctx = jnp.concatenate(parts, axis=1)
        o_ref[...] = lax.dot_general(ctx, wo_ref[...], (((1,), (0,)), ((), ())),
                                     preferred_element_type=jnp.float32)

    out = pl.pallas_call(
        body,
        out_shape=jax.ShapeDtypeStruct((SQ, DM), jnp.float32),
        in_specs=[pl.BlockSpec(memory_space=pltpu.VMEM)] * 5,
        out_specs=pl.BlockSpec(memory_space=pltpu.VMEM),
        scratch_shapes=[
            pltpu.VMEM((2, SQ, PACK), jnp.bfloat16),
            pltpu.VMEM((SQ, 128), jnp.float32),
            pltpu.SemaphoreType.DMA((2,)),
            pltpu.SemaphoreType.DMA((2,)),
        ],
        compiler_params=pltpu.CompilerParams(
            collective_id=0, vmem_limit_bytes=60 * 1024 * 1024),
    )(xb, Wqb, Kb, Vb, Wob)
    return out.reshape(1, SQ, DM)


# device time: 124303 ns/iter; 2.2479x vs baseline; 2.2479x over previous
import jax
import jax.numpy as jnp
from jax import lax
from jax.experimental import pallas as pl
from jax.experimental.pallas import tpu as pltpu

N_DEV = 4
SQ = 2048
SKV = 2048
HQ = 8
DH = 128
DM = HQ * DH
SCALE = 0.08838834764831843
QBLK = 128
N_QBLK = SQ // QBLK
PACK = DM + 128
SLAB = 4 * QBLK
N_CHUNK = 8
CHUNK = SQ // N_CHUNK
HALF = N_CHUNK // 2
TINY = 160


def kernel(x, Wq, K_ext, V_ext, Wo):
    xb = x.reshape(SQ, DM).astype(jnp.bfloat16)
    Kb = K_ext.reshape(SKV, DM).astype(jnp.bfloat16)
    Vb = V_ext.reshape(SKV, DM).astype(jnp.bfloat16)
    Wqb = Wq.astype(jnp.bfloat16)
    Wob = Wo.astype(jnp.bfloat16)

    def body(x_ref, wq_ref, k_ref, v_ref, wo_ref, o_ref,
             big, tiny, tacc, tsend, trecv, sA, sB, rX, sF, rF1, rF3):
        my = lax.axis_index("i")
        left = (my - 1) % N_DEV
        right = (my + 1) % N_DEV
        koff = my * SKV

        barrier_sem = pltpu.get_barrier_semaphore()
        for nbr in (left, right):
            pl.semaphore_signal(barrier_sem, inc=1, device_id=(nbr,),
                                device_id_type=pl.DeviceIdType.MESH)
        pl.semaphore_wait(barrier_sem, 2)

        def project_q(qstart):
            qb = lax.dot_general(
                x_ref[pl.ds(qstart, QBLK), :], wq_ref[...],
                (((1,), (0,)), ((), ())),
                preferred_element_type=jnp.float32)
            return (qb * SCALE).astype(jnp.bfloat16)

        def full_block(qstart):
            q_blk = project_q(qstart)
            qi = qstart + lax.broadcasted_iota(jnp.int32, (QBLK, SKV), 0)
            ki = koff + lax.broadcasted_iota(jnp.int32, (QBLK, SKV), 1)
            keep = (jnp.abs(qi - ki) <= 128) | (ki < 32) | (qi < 32)
            nums, ls = [], []
            for h in range(HQ):
                hc = slice(h * DH, (h + 1) * DH)
                s = lax.dot_general(q_blk[:, hc], k_ref[:, hc],
                                    (((1,), (1,)), ((), ())),
                                    preferred_element_type=jnp.float32)
                w = jnp.where(keep, jnp.exp(s), 0.0)
                num = lax.dot_general(w.astype(jnp.bfloat16), v_ref[:, hc],
                                      (((1,), (0,)), ((), ())),
                                      preferred_element_type=jnp.float32)
                nums.append(num)
                ls.append(jnp.sum(w, axis=1, keepdims=True))
            return nums, ls

        def pack_l(ls):
            return jnp.concatenate(
                ls + [jnp.zeros((QBLK, 128 - HQ), jnp.float32)], axis=1)

        nums0, ls0 = full_block(0)
        for h in range(HQ):
            tiny[0, 0:32, h * DH:(h + 1) * DH] = (
                nums0[h][0:32].astype(jnp.bfloat16))
        tiny[0, 0:32, DM:PACK] = pack_l(ls0)[0:32].astype(jnp.bfloat16)
        nums15, ls15 = full_block(SQ - QBLK)
        for h in range(HQ):
            tiny[0, 32:TINY, h * DH:(h + 1) * DH] = (
                nums15[h].astype(jnp.bfloat16))
        tiny[0, 32:TINY, DM:PACK] = pack_l(ls15).astype(jnp.bfloat16)

        tacc[...] = tiny[0].astype(jnp.float32)
        for hop in range(N_DEV - 1):
            ss, rs = hop % 2, (hop + 1) % 2
            rdma = pltpu.make_async_remote_copy(
                src_ref=tiny.at[ss], dst_ref=tiny.at[rs],
                send_sem=tsend.at[ss], recv_sem=trecv.at[rs],
                device_id=(right,), device_id_type=pl.DeviceIdType.MESH)
            rdma.start()
            rdma.wait()
            tacc[...] = tacc[...] + tiny[rs].astype(jnp.float32)

        @pl.when(my == 0)
        def _():
            for h in range(HQ):
                big[0:QBLK, h * DH:(h + 1) * DH] = (
                    (nums0[h] / ls0[h]).astype(jnp.bfloat16))

            def band_block(b, carry):
                qstart = b * QBLK
                sb = jnp.minimum(QBLK * (b - 1), SKV - 3 * QBLK)
                q_blk = project_q(qstart)
                qi = qstart + lax.broadcasted_iota(
                    jnp.int32, (QBLK, SLAB), 0)
                c0 = lax.broadcasted_iota(jnp.int32, (QBLK, QBLK), 1)
                cb = sb + lax.broadcasted_iota(
                    jnp.int32, (QBLK, 3 * QBLK), 1)
                kcols = jnp.concatenate([c0, cb], axis=1)
                keep = ((jnp.abs(qi - kcols) <= 128) | (kcols < 32)
                        | (qi < 32))
                seg0 = lax.broadcasted_iota(jnp.int32, (QBLK, SLAB), 1) < QBLK
                keep = keep & jnp.logical_not(seg0 & (sb == 0))
                for h in range(HQ):
                    hc = slice(h * DH, (h + 1) * DH)
                    ksl = jnp.concatenate(
                        [k_ref[0:QBLK, hc], k_ref[pl.ds(sb, 3 * QBLK), hc]],
                        axis=0)
                    vsl = jnp.concatenate(
                        [v_ref[0:QBLK, hc], v_ref[pl.ds(sb, 3 * QBLK), hc]],
                        axis=0)
                    s = lax.dot_general(q_blk[:, hc], ksl,
                                        (((1,), (1,)), ((), ())),
                                        preferred_element_type=jnp.float32)
                    w = jnp.where(keep, jnp.exp(s), 0.0)
                    l = jnp.sum(w, axis=1, keepdims=True)
                    num = lax.dot_general(w.astype(jnp.bfloat16), vsl,
                                          (((1,), (0,)), ((), ())),
                                          preferred_element_type=jnp.float32)
                    big[pl.ds(qstart, QBLK), hc] = (
                        (num / l).astype(jnp.bfloat16))
                return carry

            sends = []
            for c in range(N_CHUNK):
                lo, hi = max(2 * c, 1), min(2 * c + 2, N_QBLK - 1)
                lax.fori_loop(lo, hi, band_block, 0)
                rows = pl.ds(c * CHUNK, CHUNK)
                for tgt, ssem in ((1, sA), (3, sB)):
                    r = pltpu.make_async_remote_copy(
                        src_ref=big.at[rows], dst_ref=big.at[rows],
                        send_sem=ssem.at[c], recv_sem=rX.at[c],
                        device_id=(tgt,),
                        device_id_type=pl.DeviceIdType.MESH)
                    r.start()
                    sends.append(r)
            for r in sends:
                r.wait_send()

        @pl.when(my == 1)
        def _():
            fwds = []
            for c in range(N_CHUNK):
                rows = pl.ds(c * CHUNK, CHUNK)
                pltpu.make_async_remote_copy(
                    src_ref=big.at[rows], dst_ref=big.at[rows],
                    send_sem=sA.at[c], recv_sem=rX.at[c],
                    device_id=(0,),
                    device_id_type=pl.DeviceIdType.MESH).wait_recv()
                if c < HALF:
                    f = pltpu.make_async_remote_copy(
                        src_ref=big.at[rows], dst_ref=big.at[rows],
                        send_sem=sF.at[c], recv_sem=rF1.at[c],
                        device_id=(2,),
                        device_id_type=pl.DeviceIdType.MESH)
                    f.start()
                    fwds.append(f)
            for f in fwds:
                f.wait_send()

        @pl.when(my == 3)
        def _():
            fwds = []
            for c in range(N_CHUNK):
                rows = pl.ds(c * CHUNK, CHUNK)
                pltpu.make_async_remote_copy(
                    src_ref=big.at[rows], dst_ref=big.at[rows],
                    send_sem=sB.at[c], recv_sem=rX.at[c],
                    device_id=(0,),
                    device_id_type=pl.DeviceIdType.MESH).wait_recv()
                if c >= HALF:
                    f = pltpu.make_async_remote_copy(
                        src_ref=big.at[rows], dst_ref=big.at[rows],
                        send_sem=sF.at[c - HALF], recv_sem=rF3.at[c - HALF],
                        device_id=(2,),
                        device_id_type=pl.DeviceIdType.MESH)
                    f.start()
                    fwds.append(f)
            for f in fwds:
                f.wait_send()

        @pl.when(my == 2)
        def _():
            for c in range(N_CHUNK):
                rows = pl.ds(c * CHUNK, CHUNK)
                src_dev = 1 if c < HALF else 3
                rsem = rF1.at[c] if c < HALF else rF3.at[c - HALF]
                pltpu.make_async_remote_copy(
                    src_ref=big.at[rows], dst_ref=big.at[rows],
                    send_sem=sF.at[c % HALF], recv_sem=rsem,
                    device_id=(src_dev,),
                    device_id_type=pl.DeviceIdType.MESH).wait_recv()

        for h in range(HQ):
            hc = slice(h * DH, (h + 1) * DH)
            ctx_sp = (tacc[:, hc] / tacc[:, DM + h:DM + h + 1]).astype(
                jnp.bfloat16)
            big[0:32, hc] = ctx_sp[0:32]
            big[SQ - QBLK:SQ, hc] = ctx_sp[32:TINY]
        o_ref[...] = lax.dot_general(big[...], wo_ref[...],
                                     (((1,), (0,)), ((), ())),
                                     preferred_element_type=jnp.float32)

    out = pl.pallas_call(
        body,
        out_shape=jax.ShapeDtypeStruct((SQ, DM), jnp.float32),
        in_specs=[pl.BlockSpec(memory_space=pltpu.VMEM)] * 5,
        out_specs=pl.BlockSpec(memory_space=pltpu.VMEM),
        scratch_shapes=[
            pltpu.VMEM((SQ, DM), jnp.bfloat16),
            pltpu.VMEM((2, TINY, PACK), jnp.bfloat16),
            pltpu.VMEM((TINY, PACK), jnp.float32),
            pltpu.SemaphoreType.DMA((2,)),
            pltpu.SemaphoreType.DMA((2,)),
            pltpu.SemaphoreType.DMA((N_CHUNK,)),
            pltpu.SemaphoreType.DMA((N_CHUNK,)),
            pltpu.SemaphoreType.DMA((N_CHUNK,)),
            pltpu.SemaphoreType.DMA((HALF,)),
            pltpu.SemaphoreType.DMA((HALF,)),
            pltpu.SemaphoreType.DMA((HALF,)),
        ],
        compiler_params=pltpu.CompilerParams(
            collective_id=0, vmem_limit_bytes=60 * 1024 * 1024),
    )(xb, Wqb, Kb, Vb, Wob)
    return out.reshape(1, SQ, DM)


# device time: 117923 ns/iter; 2.3695x vs baseline; 1.0541x over previous
import jax
import jax.numpy as jnp
from jax import lax
from jax.experimental import pallas as pl
from jax.experimental.pallas import tpu as pltpu

N_DEV = 4
SQ = 2048
SKV = 2048
HQ = 8
DH = 128
DM = HQ * DH
SCALE = 0.08838834764831843
QBLK = 128
N_QBLK = SQ // QBLK
PACK = DM + 128
SLAB = 4 * QBLK
N_CHUNK = 8
CHUNK = SQ // N_CHUNK
HALF = N_CHUNK // 2
TINY = 160

SEGS = ((1, 6, (1, 2)), ((6), 11, (3, 4)), (11, 15, (5, 6)))
ORDER = (1, 2, 3, 4, 5, 6, 0, 7)


def kernel(x, Wq, K_ext, V_ext, Wo):
    xb = x.reshape(SQ, DM).astype(jnp.bfloat16)
    Kb = K_ext.reshape(SKV, DM).astype(jnp.bfloat16)
    Vb = V_ext.reshape(SKV, DM).astype(jnp.bfloat16)
    Wqb = Wq.astype(jnp.bfloat16)
    Wob = Wo.astype(jnp.bfloat16)

    def body(x_ref, wq_ref, k_ref, v_ref, wo_ref, o_ref,
             big, tiny, tacc, tsend, trecv, sA, sB, rX, sF, rF1, rF3):
        my = lax.axis_index("i")
        left = (my - 1) % N_DEV
        right = (my + 1) % N_DEV
        koff = my * SKV

        barrier_sem = pltpu.get_barrier_semaphore()
        for nbr in (left, right):
            pl.semaphore_signal(barrier_sem, inc=1, device_id=(nbr,),
                                device_id_type=pl.DeviceIdType.MESH)
        pl.semaphore_wait(barrier_sem, 2)

        def project_q(qstart):
            qb = lax.dot_general(
                x_ref[pl.ds(qstart, QBLK), :], wq_ref[...],
                (((1,), (0,)), ((), ())),
                preferred_element_type=jnp.float32)
            return (qb * SCALE).astype(jnp.bfloat16)

        def full_block(qstart):
            q_blk = project_q(qstart)
            qi = qstart + lax.broadcasted_iota(jnp.int32, (QBLK, SKV), 0)
            ki = koff + lax.broadcasted_iota(jnp.int32, (QBLK, SKV), 1)
            keep = (jnp.abs(qi - ki) <= 128) | (ki < 32) | (qi < 32)
            nums, ls = [], []
            for h in range(HQ):
                hc = slice(h * DH, (h + 1) * DH)
                s = lax.dot_general(q_blk[:, hc], k_ref[:, hc],
                                    (((1,), (1,)), ((), ())),
                                    preferred_element_type=jnp.float32)
                w = jnp.where(keep, jnp.exp(s), 0.0)
                num = lax.dot_general(w.astype(jnp.bfloat16), v_ref[:, hc],
                                      (((1,), (0,)), ((), ())),
                                      preferred_element_type=jnp.float32)
                nums.append(num)
                ls.append(jnp.sum(w, axis=1, keepdims=True))
            return nums, ls

        def pack_l(ls):
            return jnp.concatenate(
                ls + [jnp.zeros((QBLK, 128 - HQ), jnp.float32)], axis=1)

        nums0, ls0 = full_block(0)
        for h in range(HQ):
            tiny[0, 0:32, h * DH:(h + 1) * DH] = (
                nums0[h][0:32].astype(jnp.bfloat16))
        tiny[0, 0:32, DM:PACK] = pack_l(ls0)[0:32].astype(jnp.bfloat16)
        nums15, ls15 = full_block(SQ - QBLK)
        for h in range(HQ):
            tiny[0, 32:TINY, h * DH:(h + 1) * DH] = (
                nums15[h].astype(jnp.bfloat16))
        tiny[0, 32:TINY, DM:PACK] = pack_l(ls15).astype(jnp.bfloat16)
        tacc[...] = tiny[0].astype(jnp.float32)

        def band_block(b, carry):
            qstart = b * QBLK
            sb = jnp.minimum(QBLK * (b - 1), SKV - 3 * QBLK)
            q_blk = project_q(qstart)
            qi = qstart + lax.broadcasted_iota(jnp.int32, (QBLK, SLAB), 0)
            c0 = lax.broadcasted_iota(jnp.int32, (QBLK, QBLK), 1)
            cb = sb + lax.broadcasted_iota(jnp.int32, (QBLK, 3 * QBLK), 1)
            kcols = jnp.concatenate([c0, cb], axis=1)
            keep = (jnp.abs(qi - kcols) <= 128) | (kcols < 32) | (qi < 32)
            seg0 = lax.broadcasted_iota(jnp.int32, (QBLK, SLAB), 1) < QBLK
            keep = keep & jnp.logical_not(seg0 & (sb == 0))
            for h in range(HQ):
                hc = slice(h * DH, (h + 1) * DH)
                ksl = jnp.concatenate(
                    [k_ref[0:QBLK, hc], k_ref[pl.ds(sb, 3 * QBLK), hc]],
                    axis=0)
                vsl = jnp.concatenate(
                    [v_ref[0:QBLK, hc], v_ref[pl.ds(sb, 3 * QBLK), hc]],
                    axis=0)
                s = lax.dot_general(q_blk[:, hc], ksl,
                                    (((1,), (1,)), ((), ())),
                                    preferred_element_type=jnp.float32)
                w = jnp.where(keep, jnp.exp(s), 0.0)
                l = jnp.sum(w, axis=1, keepdims=True)
                num = lax.dot_general(w.astype(jnp.bfloat16), vsl,
                                      (((1,), (0,)), ((), ())),
                                      preferred_element_type=jnp.float32)
                big[pl.ds(qstart, QBLK), hc] = (num / l).astype(jnp.bfloat16)
            return carry

        def chunk_rdma(c, tgt, ssem):
            rows = pl.ds(c * CHUNK, CHUNK)
            return pltpu.make_async_remote_copy(
                src_ref=big.at[rows], dst_ref=big.at[rows],
                send_sem=ssem.at[c], recv_sem=rX.at[c],
                device_id=(tgt,), device_id_type=pl.DeviceIdType.MESH)

        for hop in range(N_DEV - 1):
            ss, rs = hop % 2, (hop + 1) % 2
            rdma_t = pltpu.make_async_remote_copy(
                src_ref=tiny.at[ss], dst_ref=tiny.at[rs],
                send_sem=tsend.at[ss], recv_sem=trecv.at[rs],
                device_id=(right,), device_id_type=pl.DeviceIdType.MESH)
            rdma_t.start()
            lo, hi, chunks = SEGS[hop]

            @pl.when(my == 0)
            def _(hop=hop, lo=lo, hi=hi, chunks=chunks):
                if hop == 0:
                    for h in range(HQ):
                        big[0:QBLK, h * DH:(h + 1) * DH] = (
                            (nums0[h] / ls0[h]).astype(jnp.bfloat16))
                lax.fori_loop(lo, hi, band_block, 0)
                for c in chunks:
                    chunk_rdma(c, 1, sA).start()
                    chunk_rdma(c, 3, sB).start()

            rdma_t.wait()
            tacc[...] = tacc[...] + tiny[rs].astype(jnp.float32)

        @pl.when(my == 0)
        def _():
            for h in range(HQ):
                hc = slice(h * DH, (h + 1) * DH)
                ctx_sp = (tacc[:, hc] / tacc[:, DM + h:DM + h + 1]).astype(
                    jnp.bfloat16)
                big[0:32, hc] = ctx_sp[0:32]
                big[SQ - QBLK:SQ, hc] = ctx_sp[32:TINY]
            chunk_rdma(0, 1, sA).start()
            chunk_rdma(7, 3, sB).start()
            chunk_rdma(7, 1, sA).start()
            chunk_rdma(0, 3, sB).start()
            for c in range(N_CHUNK):
                chunk_rdma(c, 1, sA).wait_send()
                chunk_rdma(c, 3, sB).wait_send()
            o_ref[...] = lax.dot_general(big[...], wo_ref[...],
                                         (((1,), (0,)), ((), ())),
                                         preferred_element_type=jnp.float32)

        def wo_chunk(c):
            rows = pl.ds(c * CHUNK, CHUNK)
            o_ref[rows, :] = lax.dot_general(
                big[rows, :], wo_ref[...], (((1,), (0,)), ((), ())),
                preferred_element_type=jnp.float32)

        @pl.when(my == 1)
        def _():
            for c in ORDER:
                chunk_rdma(c, 0, sA).wait_recv()
                if c < HALF:
                    pltpu.make_async_remote_copy(
                        src_ref=big.at[pl.ds(c * CHUNK, CHUNK)],
                        dst_ref=big.at[pl.ds(c * CHUNK, CHUNK)],
                        send_sem=sF.at[c], recv_sem=rF1.at[c],
                        device_id=(2,),
                        device_id_type=pl.DeviceIdType.MESH).start()
                wo_chunk(c)
            for c in range(HALF):
                pltpu.make_async_remote_copy(
                    src_ref=big.at[pl.ds(c * CHUNK, CHUNK)],
                    dst_ref=big.at[pl.ds(c * CHUNK, CHUNK)],
                    send_sem=sF.at[c], recv_sem=rF1.at[c],
                    device_id=(2,),
                    device_id_type=pl.DeviceIdType.MESH).wait_send()

        @pl.when(my == 3)
        def _():
            for c in ORDER:
                chunk_rdma(c, 0, sB).wait_recv()
                if c >= HALF:
                    pltpu.make_async_remote_copy(
                        src_ref=big.at[pl.ds(c * CHUNK, CHUNK)],
                        dst_ref=big.at[pl.ds(c * CHUNK, CHUNK)],
                        send_sem=sF.at[c - HALF], recv_sem=rF3.at[c - HALF],
                        device_id=(2,),
                        device_id_type=pl.DeviceIdType.MESH).start()
                wo_chunk(c)
            for c in range(HALF):
                pltpu.make_async_remote_copy(
                    src_ref=big.at[pl.ds((c + HALF) * CHUNK, CHUNK)],
                    dst_ref=big.at[pl.ds((c + HALF) * CHUNK, CHUNK)],
                    send_sem=sF.at[c], recv_sem=rF3.at[c],
                    device_id=(2,),
                    device_id_type=pl.DeviceIdType.MESH).wait_send()

        @pl.when(my == 2)
        def _():
            for c in ORDER:
                rows = pl.ds(c * CHUNK, CHUNK)
                src_dev = 1 if c < HALF else 3
                rsem = rF1.at[c] if c < HALF else rF3.at[c - HALF]
                pltpu.make_async_remote_copy(
                    src_ref=big.at[rows], dst_ref=big.at[rows],
                    send_sem=sF.at[c % HALF], recv_sem=rsem,
                    device_id=(src_dev,),
                    device_id_type=pl.DeviceIdType.MESH).wait_recv()
                wo_chunk(c)

    out = pl.pallas_call(
        body,
        out_shape=jax.ShapeDtypeStruct((SQ, DM), jnp.float32),
        in_specs=[pl.BlockSpec(memory_space=pltpu.VMEM)] * 5,
        out_specs=pl.BlockSpec(memory_space=pltpu.VMEM),
        scratch_shapes=[
            pltpu.VMEM((SQ, DM), jnp.bfloat16),
            pltpu.VMEM((2, TINY, PACK), jnp.bfloat16),
            pltpu.VMEM((TINY, PACK), jnp.float32),
            pltpu.SemaphoreType.DMA((2,)),
            pltpu.SemaphoreType.DMA((2,)),
            pltpu.SemaphoreType.DMA((N_CHUNK,)),
            pltpu.SemaphoreType.DMA((N_CHUNK,)),
            pltpu.SemaphoreType.DMA((N_CHUNK,)),
            pltpu.SemaphoreType.DMA((HALF,)),
            pltpu.SemaphoreType.DMA((HALF,)),
            pltpu.SemaphoreType.DMA((HALF,)),
        ],
        compiler_params=pltpu.CompilerParams(
            collective_id=0, vmem_limit_bytes=60 * 1024 * 1024),
    )(xb, Wqb, Kb, Vb, Wob)
    return out.reshape(1, SQ, DM)
